# baseline (device time: 24503 ns/iter reference)
import jax
import jax.numpy as jnp
from jax import lax
from jax.experimental import pallas as pl
from jax.experimental.pallas import tpu as pltpu

N_DEV = 8
NS = 4


def kernel(x):
    m, n = x.shape
    mc = m // N_DEV
    nw = n // NS

    def body(x_ref, out_ref, x16_ref, rs_buf, p1_send, p1_recv, p2_send, p2_recv):
        my_i = lax.axis_index("i")

        x16_ref[:, :] = x_ref[:, :].astype(jnp.bfloat16)

        p1 = []
        for s in range(NS):
            for o in range(1, N_DEV):
                t = (my_i + o) % N_DEV
                k = o - 1
                rdma = pltpu.make_async_remote_copy(
                    src_ref=x16_ref.at[pl.ds(t * mc, mc), pl.ds(s * nw, nw)],
                    dst_ref=rs_buf.at[s, k],
                    send_sem=p1_send.at[s, k],
                    recv_sem=p1_recv.at[s, k],
                    device_id=(t,),
                    device_id_type=pl.DeviceIdType.MESH,
                )
                rdma.start()
                p1.append(rdma)

        p2 = []
        for s in range(NS):
            acc = x_ref[pl.ds(my_i * mc, mc), pl.ds(s * nw, nw)]
            for k in range(N_DEV - 1):
                p1[s * (N_DEV - 1) + k].wait_recv()
                acc = acc + rs_buf[s, k, :, :].astype(jnp.float32)
            out_ref[pl.ds(my_i * mc, mc), pl.ds(s * nw, nw)] = acc.astype(
                jnp.bfloat16
            )
            for o in range(1, N_DEV):
                t = (my_i + o) % N_DEV
                k = o - 1
                rdma = pltpu.make_async_remote_copy(
                    src_ref=out_ref.at[pl.ds(my_i * mc, mc), pl.ds(s * nw, nw)],
                    dst_ref=out_ref.at[pl.ds(my_i * mc, mc), pl.ds(s * nw, nw)],
                    send_sem=p2_send.at[s, k],
                    recv_sem=p2_recv.at[s, k],
                    device_id=(t,),
                    device_id_type=pl.DeviceIdType.MESH,
                )
                rdma.start()
                p2.append(rdma)

        for r in p1:
            r.wait_send()
        for r in p2:
            r.wait_recv()
        for r in p2:
            r.wait_send()

    out_shape = jax.ShapeDtypeStruct((m, n), jnp.bfloat16)
    return pl.pallas_call(
        body,
        out_shape=out_shape,
        in_specs=[pl.BlockSpec(memory_space=pltpu.VMEM)],
        out_specs=pl.BlockSpec(memory_space=pltpu.VMEM),
        scratch_shapes=[
            pltpu.VMEM((m, n), jnp.bfloat16),
            pltpu.VMEM((NS, N_DEV - 1, mc, nw), jnp.bfloat16),
            pltpu.SemaphoreType.DMA((NS, N_DEV - 1)),
            pltpu.SemaphoreType.DMA((NS, N_DEV - 1)),
            pltpu.SemaphoreType.DMA((NS, N_DEV - 1)),
            pltpu.SemaphoreType.DMA((NS, N_DEV - 1)),
        ],
    )(x)


# device time: 20488 ns/iter; 1.1960x vs baseline; 1.1960x over previous
import jax
import jax.numpy as jnp
from jax import lax
from jax.experimental import pallas as pl
from jax.experimental.pallas import tpu as pltpu

N_DEV = 8
NS = 2

SEND_ORDER = [6, 2, 5, 7, 1, 3, 4]
WAIT_ORDER = [1, 3, 4, 2, 5, 7, 6]
SLOT = {g: i for i, g in enumerate(SEND_ORDER)}


def kernel(x):
    m, n = x.shape
    mc = m // N_DEV
    nw = n // NS

    def body(x_ref, out_ref, x16_ref, rs_buf, p1_send, p1_recv, p2_send, p2_recv):
        my_i = lax.axis_index("i")

        barrier_sem = pltpu.get_barrier_semaphore()
        for g in SEND_ORDER:
            pl.semaphore_signal(
                barrier_sem, inc=1, device_id=(my_i ^ g,),
                device_id_type=pl.DeviceIdType.MESH,
            )
        x16_ref[:, :] = x_ref[:, :].astype(jnp.bfloat16)
        pl.semaphore_wait(barrier_sem, N_DEV - 1)

        p1 = {}
        for s in range(NS):
            for g in SEND_ORDER:
                t = my_i ^ g
                k = SLOT[g]
                rdma = pltpu.make_async_remote_copy(
                    src_ref=x16_ref.at[pl.ds(t * mc, mc), pl.ds(s * nw, nw)],
                    dst_ref=rs_buf.at[s, k],
                    send_sem=p1_send.at[s, k],
                    recv_sem=p1_recv.at[s, k],
                    device_id=(t,),
                    device_id_type=pl.DeviceIdType.MESH,
                )
                rdma.start()
                p1[(s, g)] = rdma
        p2 = {}
        for s in range(NS):
            acc = x_ref[pl.ds(my_i * mc, mc), pl.ds(s * nw, nw)]
            for g in WAIT_ORDER:
                p1[(s, g)].wait_recv()
                acc = acc + rs_buf[s, SLOT[g], :, :].astype(jnp.float32)
            out_ref[pl.ds(my_i * mc, mc), pl.ds(s * nw, nw)] = acc.astype(
                jnp.bfloat16
            )
            for g in SEND_ORDER:
                t = my_i ^ g
                k = SLOT[g]
                rdma = pltpu.make_async_remote_copy(
                    src_ref=out_ref.at[pl.ds(my_i * mc, mc), pl.ds(s * nw, nw)],
                    dst_ref=out_ref.at[pl.ds(my_i * mc, mc), pl.ds(s * nw, nw)],
                    send_sem=p2_send.at[s, k],
                    recv_sem=p2_recv.at[s, k],
                    device_id=(t,),
                    device_id_type=pl.DeviceIdType.MESH,
                )
                rdma.start()
                p2[(s, g)] = rdma

        for r in p1.values():
            r.wait_send()
        for r in p2.values():
            r.wait_recv()
        for r in p2.values():
            r.wait_send()

    out_shape = jax.ShapeDtypeStruct((m, n), jnp.bfloat16)
    return pl.pallas_call(
        body,
        out_shape=out_shape,
        in_specs=[pl.BlockSpec(memory_space=pltpu.VMEM)],
        out_specs=pl.BlockSpec(memory_space=pltpu.VMEM),
        scratch_shapes=[
            pltpu.VMEM((m, n), jnp.bfloat16),
            pltpu.VMEM((NS, N_DEV - 1, mc, nw), jnp.bfloat16),
            pltpu.SemaphoreType.DMA((NS, N_DEV - 1)),
            pltpu.SemaphoreType.DMA((NS, N_DEV - 1)),
            pltpu.SemaphoreType.DMA((NS, N_DEV - 1)),
            pltpu.SemaphoreType.DMA((NS, N_DEV - 1)),
        ],
        compiler_params=pltpu.CompilerParams(collective_id=0),
    )(x)
